# Initial kernel scaffold; baseline (speedup 1.0000x reference)
import jax
import jax.numpy as jnp
from jax.experimental import pallas as pl


def kernel(x, edge_index, W1, b1, W2, b2, W3, b3, W4, b4, W5, b5, W6, b6, gamma1, beta1, gamma2, beta2, gamma3, beta3, gamma4, beta4, gamma5, beta5):
    raise NotImplementedError("write your pallas kernel here")


# speedup vs baseline: 6.2693x; 6.2693x over previous
import functools

import jax
import jax.numpy as jnp
from jax import lax
from jax.experimental import pallas as pl
from jax.experimental.pallas import tpu as pltpu
from jax.experimental.pallas import tpu_sc as plsc

NC, NS = 2, 16
NW = NC * NS
K = 80
DW = 16


def _rup(v, m):
    return -(-v // m) * m


def _mesh():
    return plsc.VectorSubcoreMesh(
        core_axis_name="c", subcore_axis_name="s",
        num_cores=NC, num_subcores=NS)


_SC_PARAMS = pltpu.CompilerParams(use_tc_tiling_on_sc=False)


@functools.lru_cache(None)
def _make_deg(Np, E):
    epw = E // NW
    nsteps = epw // K
    rpt = Np // NS

    @functools.partial(
        pl.kernel,
        out_type=jax.ShapeDtypeStruct((NC, Np, DW), jnp.float32),
        mesh=_mesh(),
        scratch_types=[
            pltpu.VMEM((K,), jnp.int32),
            pltpu.VMEM((K, DW), jnp.float32),
            pltpu.VMEM((rpt, DW), jnp.float32),
            pltpu.VMEM_SHARED((Np, DW), jnp.float32),
            pltpu.SemaphoreType.DMA,
        ],
        compiler_params=_SC_PARAMS,
    )
    def deg(dst_h, out, didx, ones, zbuf, sh, sem):
        c = lax.axis_index("c")
        s = lax.axis_index("s")

        one_v = jnp.ones((DW,), jnp.float32)
        zero_v = jnp.zeros((DW,), jnp.float32)

        def fill_ones(i, carry):
            ones[i, :] = one_v
            return carry
        lax.fori_loop(0, K, fill_ones, 0)

        def fill_zero(i, carry):
            zbuf[i, :] = zero_v
            return carry
        lax.fori_loop(0, rpt, fill_zero, 0)

        pltpu.sync_copy(zbuf, sh.at[pl.ds(s * rpt, rpt)])
        plsc.subcore_barrier()

        base = (c * NS + s) * epw

        def step(i, carry):
            off = base + i * K
            pltpu.sync_copy(dst_h.at[pl.ds(off, K)], didx)
            pltpu.sync_copy(ones, sh.at[didx], add=True)
            return carry
        lax.fori_loop(0, nsteps, step, 0)

        plsc.subcore_barrier()
        pltpu.sync_copy(sh.at[pl.ds(s * rpt, rpt)],
                        out.at[c, pl.ds(s * rpt, rpt)])

    return deg


@functools.lru_cache(None)
def _make_agg(Np, Fh, E):
    epw = E // NS
    nsteps = epw // K
    rpt = Np // NS

    @functools.partial(
        pl.kernel,
        out_type=(jax.ShapeDtypeStruct((Np, Fh), jnp.float32),
                  jax.ShapeDtypeStruct((Np, Fh), jnp.float32)),
        mesh=_mesh(),
        scratch_types=[
            pltpu.VMEM((K,), jnp.int32),
            pltpu.VMEM((K,), jnp.int32),
            pltpu.VMEM((K, Fh), jnp.float32),
            pltpu.VMEM_SHARED((Np, Fh), jnp.float32),
            pltpu.SemaphoreType.DMA,
        ],
        compiler_params=_SC_PARAMS,
    )
    def agg(g0, g1, src_h, dst_h, out0, out1, sidx, didx, rows, sh, sem):
        c = lax.axis_index("c")
        s = lax.axis_index("s")

        def run(g, out):
            pltpu.sync_copy(g.at[pl.ds(s * rpt, rpt)],
                            sh.at[pl.ds(s * rpt, rpt)])
            plsc.subcore_barrier()
            base = s * epw

            def step(i, carry):
                off = base + i * K
                pltpu.sync_copy(src_h.at[pl.ds(off, K)], sidx)
                pltpu.sync_copy(dst_h.at[pl.ds(off, K)], didx)
                pltpu.async_copy(g.at[sidx], rows, sem).wait()
                pltpu.sync_copy(rows, sh.at[didx], add=True)
                return carry
            lax.fori_loop(0, nsteps, step, 0)

            plsc.subcore_barrier()
            pltpu.sync_copy(sh.at[pl.ds(s * rpt, rpt)],
                            out.at[pl.ds(s * rpt, rpt)])

        @pl.when(c == 0)
        def _():
            run(g0, out0)

        @pl.when(c == 1)
        def _():
            run(g1, out1)

    return agg


def _tc(body, out_shape, *args):
    return pl.pallas_call(body, out_shape=out_shape)(*args)


def _mm1_body(Fh, degp, x, w, dinv, g0, g1):
    deg = degp[0, :, 0:1] + degp[1, :, 0:1] + 1.0
    div = lax.rsqrt(deg)
    dinv[...] = div
    g = jnp.dot(x[...], w[...], preferred_element_type=jnp.float32) * div
    g0[...] = g[:, :Fh]
    g1[...] = g[:, Fh:]


def _mid_body(Nreal, Fh_out, a0, a1, dinv, b, gamma, beta, w, g0, g1):
    a = jnp.concatenate([a0[...], a1[...]], axis=1)
    div = dinv[...]
    z = jnp.maximum(a * div + b[...], 0.0)
    zn = z[:Nreal, :]
    mu = jnp.sum(zn, axis=0, keepdims=True) * (1.0 / Nreal)
    dvn = zn - mu
    var = jnp.sum(dvn * dvn, axis=0, keepdims=True) * (1.0 / Nreal)
    zb = (z - mu) * lax.rsqrt(var + 1e-5) * gamma[...] + beta[...]
    g = jnp.dot(zb, w[...], preferred_element_type=jnp.float32) * div
    g0[...] = g[:, :Fh_out]
    g1[...] = g[:, Fh_out:]


def _fin_body(C, a0, a1, dinv, b, out):
    a = jnp.concatenate([a0[...], a1[...]], axis=1)
    z = a * dinv[...] + b[...]
    col = lax.broadcasted_iota(jnp.int32, z.shape, 1)
    valid = col < C
    zm = jnp.where(valid, z, -jnp.inf)
    m = jnp.max(zm, axis=1, keepdims=True)
    e = jnp.where(valid, jnp.exp(z - m), 0.0)
    ssum = jnp.sum(e, axis=1, keepdims=True)
    out[...] = z - m - jnp.log(ssum)


def kernel(x, edge_index, W1, b1, W2, b2, W3, b3, W4, b4, W5, b5, W6, b6,
           gamma1, beta1, gamma2, beta2, gamma3, beta3, gamma4, beta4,
           gamma5, beta5):
    N = x.shape[0]
    E = edge_index.shape[1]
    C = W6.shape[1]
    Np = _rup(N, 128)
    f32 = jnp.float32

    Ws = [W1, W2, W3, W4, W5, W6]
    bs = [b1, b2, b3, b4, b5, b6]
    gammas = [gamma1, gamma2, gamma3, gamma4, gamma5]
    betas = [beta1, beta2, beta3, beta4, beta5]

    Fhs = [_rup(_rup(w.shape[1], 2) // 2, 16) for w in Ws]
    Fps = [2 * fh for fh in Fhs]

    src = edge_index[0]
    dst = edge_index[1]

    x_p = jnp.zeros((Np, x.shape[1]), f32).at[:N].set(x)

    def padw(w, rows, cols):
        return jnp.zeros((rows, cols), f32).at[:w.shape[0], :w.shape[1]].set(w)

    def padv(v, cols):
        return jnp.zeros((1, cols), f32).at[0, :v.shape[0]].set(v)

    degp = _make_deg(Np, E)(dst)
    W1_p = padw(W1, W1.shape[0], Fps[0])
    dinv, g0, g1 = _tc(
        functools.partial(_mm1_body, Fhs[0]),
        (jax.ShapeDtypeStruct((Np, 1), f32),
         jax.ShapeDtypeStruct((Np, Fhs[0]), f32),
         jax.ShapeDtypeStruct((Np, Fhs[0]), f32)),
        degp, x_p, W1_p)

    for l in range(6):
        a0, a1 = _make_agg(Np, Fhs[l], E)(g0, g1, src, dst)
        if l < 5:
            w_next = padw(Ws[l + 1], Fps[l], Fps[l + 1])
            g0, g1 = _tc(
                functools.partial(_mid_body, N, Fhs[l + 1]),
                (jax.ShapeDtypeStruct((Np, Fhs[l + 1]), f32),
                 jax.ShapeDtypeStruct((Np, Fhs[l + 1]), f32)),
                a0, a1, dinv, padv(bs[l], Fps[l]),
                padv(gammas[l], Fps[l]), padv(betas[l], Fps[l]), w_next)
        else:
            out = _tc(
                functools.partial(_fin_body, C),
                jax.ShapeDtypeStruct((Np, Fps[l]), f32),
                a0, a1, dinv, padv(bs[l], Fps[l]))

    return out[:N, :C]
